# Initial kernel scaffold; baseline (speedup 1.0000x reference)
import jax
import jax.numpy as jnp
from jax.experimental import pallas as pl


def kernel(x, edge_index, edge_attr, params):
    raise NotImplementedError("write your pallas kernel here")


# speedup vs baseline: 1.1701x; 1.1701x over previous
import functools

import jax
import jax.numpy as jnp
from jax.experimental import pallas as pl
from jax.experimental.pallas import tpu as pltpu

N = 10000
E = 320000
D = 128
H = 8
C = 16
EDIM = 3
ROWS = 1000


def _ln(x, g, b, eps=1e-5):
    mu = x.mean(-1, keepdims=True)
    var = ((x - mu) ** 2).mean(-1, keepdims=True)
    return (x - mu) * jax.lax.rsqrt(var + eps) * g + b


def _in_proj_body(x_ref, w_ref, b_ref, g_ref, beta_ref, o_ref):
    h = jnp.dot(x_ref[...], w_ref[...], preferred_element_type=jnp.float32)
    h = jax.nn.gelu(h + b_ref[...])
    o_ref[...] = _ln(h, g_ref[...], beta_ref[...])


def _lin_body(h_ref, wl_ref, bl_ref, wr_ref, br_ref, xl_ref, xr_ref):
    h = h_ref[...]
    xl_ref[...] = jnp.dot(h, wl_ref[...], preferred_element_type=jnp.float32) + bl_ref[...]
    xr_ref[...] = jnp.dot(h, wr_ref[...], preferred_element_type=jnp.float32) + br_ref[...]


def _post_body(h_ref, agg_ref, lng_ref, lnb_ref, gw1_ref, gb1_ref,
               gw2_ref, gb2_ref, o_ref):
    h = h_ref[...]
    hblk = _ln(h + agg_ref[...], lng_ref[...], lnb_ref[...])
    delta = hblk - h
    t = jax.nn.gelu(jnp.dot(delta, gw1_ref[...], preferred_element_type=jnp.float32)
                    + gb1_ref[...])
    gate = jax.nn.sigmoid(jnp.dot(t, gw2_ref[...], preferred_element_type=jnp.float32)
                          + gb2_ref[...])
    o_ref[...] = h + gate * delta


def _fin_body(h_ref, g_ref, b_ref, o_ref):
    o_ref[...] = _ln(h_ref[...], g_ref[...], b_ref[...])


def _row_spec():
    return pl.BlockSpec((ROWS, D), lambda i: (i, 0))


def _full(shape):
    return pl.BlockSpec(shape, lambda i: tuple(0 for _ in shape))


def _tc_call(body, in_specs, out_specs, out_shape, *args):
    return pl.pallas_call(
        body,
        grid=(N // ROWS,),
        in_specs=in_specs,
        out_specs=out_specs,
        out_shape=out_shape,
    )(*args)


def _in_proj(x, w, b, g, beta):
    return _tc_call(
        _in_proj_body,
        [_row_spec(), _full((D, D)), _full((1, D)), _full((1, D)), _full((1, D))],
        _row_spec(),
        jax.ShapeDtypeStruct((N, D), jnp.float32),
        x, w, b.reshape(1, D), g.reshape(1, D), beta.reshape(1, D))


def _lin(h, wl, bl, wr, br):
    return _tc_call(
        _lin_body,
        [_row_spec(), _full((D, D)), _full((1, D)), _full((D, D)), _full((1, D))],
        (_row_spec(), _row_spec()),
        (jax.ShapeDtypeStruct((N, D), jnp.float32),
         jax.ShapeDtypeStruct((N, D), jnp.float32)),
        h, wl, bl.reshape(1, D), wr, br.reshape(1, D))


def _post(h, agg, lp):
    return _tc_call(
        _post_body,
        [_row_spec(), _row_spec(), _full((1, D)), _full((1, D)),
         _full((D, D // 2)), _full((1, D // 2)), _full((D // 2, D)), _full((1, D))],
        _row_spec(),
        jax.ShapeDtypeStruct((N, D), jnp.float32),
        h, agg, lp['ln_g'].reshape(1, D), lp['ln_b'].reshape(1, D),
        lp['gW1'], lp['gb1'].reshape(1, D // 2), lp['gW2'], lp['gb2'].reshape(1, D))


def _fin(h, g, b):
    return _tc_call(
        _fin_body,
        [_row_spec(), _full((1, D)), _full((1, D))],
        _row_spec(),
        jax.ShapeDtypeStruct((N, D), jnp.float32),
        h, g.reshape(1, D), b.reshape(1, D))


def _message_pass(xl, xr, src, dst, edge_attr, lp):
    ef = (edge_attr @ lp['We'])
    m = xl[src] + xr[dst] + ef
    s = jax.nn.leaky_relu(m.reshape(E, H, C), negative_slope=0.2)
    logits = (s * lp['att'][None]).sum(-1)
    ex = jnp.exp(logits)
    denom = jax.ops.segment_sum(ex, dst, num_segments=N)
    msg = ex[:, :, None] * xl[src].reshape(E, H, C)
    num = jax.ops.segment_sum(msg, dst, num_segments=N)
    agg = num / (denom[:, :, None] + 1e-16)
    return agg.reshape(N, H * C) + lp['bias']


def kernel(x, edge_index, edge_attr, params):
    src = edge_index[0]
    dst = edge_index[1]
    h = _in_proj(x, params['in_W'], params['in_b'], params['in_g'], params['in_beta'])
    for lp in params['layers']:
        xl, xr = _lin(h, lp['Wl'], lp['bl'], lp['Wr'], lp['br'])
        agg = _message_pass(xl, xr, src, dst, edge_attr, lp)
        h = _post(h, agg, lp)
    return _fin(h, params['fn_g'], params['fn_b'])
